# baseline (speedup 1.0000x reference)
import jax
import jax.numpy as jnp
from jax import lax
from jax.experimental import pallas as pl
from jax.experimental.pallas import tpu as pltpu
from jax.experimental.pallas import tpu_sc as plsc

_D = 768
_BN = 256
_NW = 32
_GC = 64


def _sc_permute(table, src_idx, dst_idx, n_valid, n_out):
    chunks = n_valid // (_NW * _GC)
    assert n_valid == chunks * _NW * _GC
    mesh = plsc.VectorSubcoreMesh(core_axis_name="c", subcore_axis_name="s")

    def body(table_ref, sidx_ref, didx_ref, out_ref, sidx_v, didx_v,
             rows0, rows1, gsem0, gsem1, wsem0, wsem1):
        wid = lax.axis_index("s") * 2 + lax.axis_index("c")
        bufs = (rows0, rows1)
        gsems = (gsem0, gsem1)
        wsems = (wsem0, wsem1)
        pltpu.sync_copy(sidx_ref.at[pl.ds(wid * chunks, chunks)], sidx_v)
        pltpu.sync_copy(didx_ref.at[pl.ds(wid * chunks, chunks)], didx_v)

        gh = [None, None]
        wh = [None, None]
        gh[0] = pltpu.async_copy(
            table_ref.at[sidx_v.at[0]], bufs[0], gsems[0])
        for c in range(chunks):
            cb = c % 2
            nb = (c + 1) % 2
            if c + 1 < chunks:
                if wh[nb] is not None:
                    wh[nb].wait()
                    wh[nb] = None
                gh[nb] = pltpu.async_copy(
                    table_ref.at[sidx_v.at[c + 1]], bufs[nb], gsems[nb])
            gh[cb].wait()
            wh[cb] = pltpu.async_copy(
                bufs[cb], out_ref.at[didx_v.at[c]], wsems[cb])
        for h in wh:
            if h is not None:
                h.wait()

    return pl.kernel(
        body,
        out_type=jax.ShapeDtypeStruct((n_out, _D), table.dtype),
        mesh=mesh,
        scratch_types=[
            pltpu.VMEM((chunks, _GC), jnp.int32),
            pltpu.VMEM((chunks, _GC), jnp.int32),
            pltpu.VMEM((_GC, _D), jnp.float32),
            pltpu.VMEM((_GC, _D), jnp.float32),
            pltpu.SemaphoreType.DMA,
            pltpu.SemaphoreType.DMA,
            pltpu.SemaphoreType.DMA,
            pltpu.SemaphoreType.DMA,
        ],
    )(table, src_idx.reshape(_NW * chunks, _GC),
      dst_idx.reshape(_NW * chunks, _GC))


def _mm_body(gid_ref, x_ref, w_ref, b_ref, o_ref):
    i = pl.program_id(0)

    @pl.when(gid_ref[i] < 64)
    def _():
        xb = x_ref[...].astype(jnp.bfloat16)
        wb = w_ref[0].astype(jnp.bfloat16)
        o_ref[...] = (
            jnp.dot(xb, wb, preferred_element_type=jnp.float32) + b_ref[0]
        )


def _segment_matmul(xs, weight, bias, tile_gid):
    n_tiles = xs.shape[0] // _BN
    last = n_tiles - 1

    grid_spec = pltpu.PrefetchScalarGridSpec(
        num_scalar_prefetch=1,
        grid=(n_tiles,),
        in_specs=[
            pl.BlockSpec(
                (_BN, _D), lambda i, g: (jnp.where(g[i] < 64, i, last), 0)),
            pl.BlockSpec((1, _D, _D), lambda i, g: (g[i] & 63, 0, 0)),
            pl.BlockSpec((1, 1, _D), lambda i, g: (g[i] & 63, 0, 0)),
        ],
        out_specs=pl.BlockSpec(
            (_BN, _D), lambda i, g: (jnp.where(g[i] < 64, i, last), 0)),
    )
    return pl.pallas_call(
        _mm_body,
        grid_spec=grid_spec,
        out_shape=jax.ShapeDtypeStruct((xs.shape[0], _D), jnp.float32),
    )(tile_gid, xs, weight, bias[:, None, :])


def kernel(x, type_vec, weight, bias):
    n = x.shape[0]
    t = weight.shape[0]
    n_pad = n + t * _BN

    counts = jnp.zeros((t,), jnp.int32).at[type_vec].add(1)
    group_start = jnp.cumsum(counts) - counts
    padded_counts = ((counts + _BN - 1) // _BN) * _BN
    padded_start = jnp.cumsum(padded_counts) - padded_counts
    perm = jnp.argsort(type_vec).astype(jnp.int32)
    sorted_types = type_vec[perm]
    delta = padded_start - group_start
    ppos = jnp.arange(n, dtype=jnp.int32) + delta[sorted_types]
    n_tiles = n_pad // _BN
    tile_base = jnp.arange(n_tiles, dtype=jnp.int32) * _BN
    tile_gid = (
        jnp.sum(padded_start[None, :] <= tile_base[:, None], axis=1,
                dtype=jnp.int32) - 1
    )
    occupied_tiles = jnp.sum(padded_counts, dtype=jnp.int32) // _BN
    tile_gid = tile_gid + 64 * (
        jnp.arange(n_tiles, dtype=jnp.int32) >= occupied_tiles
    ).astype(jnp.int32)
    tile_gid = tile_gid + jnp.minimum(ppos[0], 0)

    xs = _sc_permute(x, perm, ppos, n, n_pad)
    ys = _segment_matmul(xs, weight, bias, tile_gid)
    return _sc_permute(ys, ppos, perm, n, n)


# speedup vs baseline: 1.0812x; 1.0812x over previous
import jax
import jax.numpy as jnp
from jax import lax
from jax.experimental import pallas as pl
from jax.experimental.pallas import tpu as pltpu
from jax.experimental.pallas import tpu_sc as plsc

_D = 768
_BN = 512
_NW = 32
_GC = 64


def _sc_permute(table, src_idx, dst_idx, n_valid, n_out):
    chunks = n_valid // (_NW * _GC)
    assert n_valid == chunks * _NW * _GC
    mesh = plsc.VectorSubcoreMesh(core_axis_name="c", subcore_axis_name="s")

    def body(table_ref, sidx_ref, didx_ref, out_ref, sidx_v, didx_v,
             rows0, rows1, gsem0, gsem1, wsem0, wsem1):
        wid = lax.axis_index("s") * 2 + lax.axis_index("c")
        bufs = (rows0, rows1)
        gsems = (gsem0, gsem1)
        wsems = (wsem0, wsem1)
        pltpu.sync_copy(sidx_ref.at[pl.ds(wid * chunks, chunks)], sidx_v)
        pltpu.sync_copy(didx_ref.at[pl.ds(wid * chunks, chunks)], didx_v)

        gh = [None, None]
        wh = [None, None]
        gh[0] = pltpu.async_copy(
            table_ref.at[sidx_v.at[0]], bufs[0], gsems[0])
        for c in range(chunks):
            cb = c % 2
            nb = (c + 1) % 2
            if c + 1 < chunks:
                if wh[nb] is not None:
                    wh[nb].wait()
                    wh[nb] = None
                gh[nb] = pltpu.async_copy(
                    table_ref.at[sidx_v.at[c + 1]], bufs[nb], gsems[nb])
            gh[cb].wait()
            wh[cb] = pltpu.async_copy(
                bufs[cb], out_ref.at[didx_v.at[c]], wsems[cb])
        for h in wh:
            if h is not None:
                h.wait()

    return pl.kernel(
        body,
        out_type=jax.ShapeDtypeStruct((n_out, _D), table.dtype),
        mesh=mesh,
        scratch_types=[
            pltpu.VMEM((chunks, _GC), jnp.int32),
            pltpu.VMEM((chunks, _GC), jnp.int32),
            pltpu.VMEM((_GC, _D), jnp.float32),
            pltpu.VMEM((_GC, _D), jnp.float32),
            pltpu.SemaphoreType.DMA,
            pltpu.SemaphoreType.DMA,
            pltpu.SemaphoreType.DMA,
            pltpu.SemaphoreType.DMA,
        ],
    )(table, src_idx.reshape(_NW * chunks, _GC),
      dst_idx.reshape(_NW * chunks, _GC))


def _mm_body(gid_ref, x_ref, w_ref, b_ref, o_ref):
    i = pl.program_id(0)

    @pl.when(gid_ref[i] < 64)
    def _():
        xb = x_ref[...].astype(jnp.bfloat16)
        wb = w_ref[0].astype(jnp.bfloat16)
        o_ref[...] = (
            jnp.dot(xb, wb, preferred_element_type=jnp.float32) + b_ref[0]
        )


def _segment_matmul(xs, weight, bias, tile_gid):
    n_tiles = xs.shape[0] // _BN
    last = n_tiles - 1

    grid_spec = pltpu.PrefetchScalarGridSpec(
        num_scalar_prefetch=1,
        grid=(n_tiles,),
        in_specs=[
            pl.BlockSpec(
                (_BN, _D), lambda i, g: (jnp.where(g[i] < 64, i, last), 0)),
            pl.BlockSpec((1, _D, _D), lambda i, g: (g[i] & 63, 0, 0)),
            pl.BlockSpec((1, 1, _D), lambda i, g: (g[i] & 63, 0, 0)),
        ],
        out_specs=pl.BlockSpec(
            (_BN, _D), lambda i, g: (jnp.where(g[i] < 64, i, last), 0)),
    )
    return pl.pallas_call(
        _mm_body,
        grid_spec=grid_spec,
        out_shape=jax.ShapeDtypeStruct((xs.shape[0], _D), jnp.float32),
    )(tile_gid, xs, weight, bias[:, None, :])


def kernel(x, type_vec, weight, bias):
    n = x.shape[0]
    t = weight.shape[0]
    n_pad = n + t * _BN

    counts = jnp.zeros((t,), jnp.int32).at[type_vec].add(1)
    group_start = jnp.cumsum(counts) - counts
    padded_counts = ((counts + _BN - 1) // _BN) * _BN
    padded_start = jnp.cumsum(padded_counts) - padded_counts
    perm = jnp.argsort(type_vec).astype(jnp.int32)
    sorted_types = type_vec[perm]
    delta = padded_start - group_start
    ppos = jnp.arange(n, dtype=jnp.int32) + delta[sorted_types]
    n_tiles = n_pad // _BN
    tile_base = jnp.arange(n_tiles, dtype=jnp.int32) * _BN
    tile_gid = (
        jnp.sum(padded_start[None, :] <= tile_base[:, None], axis=1,
                dtype=jnp.int32) - 1
    )
    occupied_tiles = jnp.sum(padded_counts, dtype=jnp.int32) // _BN
    tile_gid = tile_gid + 64 * (
        jnp.arange(n_tiles, dtype=jnp.int32) >= occupied_tiles
    ).astype(jnp.int32)

    xs = _sc_permute(x, perm, ppos, n, n_pad)
    ys = _segment_matmul(xs, weight, bias, tile_gid)
    return _sc_permute(ys, ppos, perm, n, n)


# speedup vs baseline: 1.0980x; 1.0155x over previous
import jax
import jax.numpy as jnp
from jax import lax
from jax.experimental import pallas as pl
from jax.experimental.pallas import tpu as pltpu
from jax.experimental.pallas import tpu_sc as plsc

_D = 768
_BN = 1024
_NW = 32
_GC = 64


def _sc_permute(table, src_idx, dst_idx, n_valid, n_out):
    chunks = n_valid // (_NW * _GC)
    assert n_valid == chunks * _NW * _GC
    mesh = plsc.VectorSubcoreMesh(core_axis_name="c", subcore_axis_name="s")

    def body(table_ref, sidx_ref, didx_ref, out_ref, sidx_v, didx_v,
             rows0, rows1, gsem0, gsem1, wsem0, wsem1):
        wid = lax.axis_index("s") * 2 + lax.axis_index("c")
        bufs = (rows0, rows1)
        gsems = (gsem0, gsem1)
        wsems = (wsem0, wsem1)
        pltpu.sync_copy(sidx_ref.at[pl.ds(wid * chunks, chunks)], sidx_v)
        pltpu.sync_copy(didx_ref.at[pl.ds(wid * chunks, chunks)], didx_v)

        gh = [None, None]
        wh = [None, None]
        gh[0] = pltpu.async_copy(
            table_ref.at[sidx_v.at[0]], bufs[0], gsems[0])
        for c in range(chunks):
            cb = c % 2
            nb = (c + 1) % 2
            if c + 1 < chunks:
                if wh[nb] is not None:
                    wh[nb].wait()
                    wh[nb] = None
                gh[nb] = pltpu.async_copy(
                    table_ref.at[sidx_v.at[c + 1]], bufs[nb], gsems[nb])
            gh[cb].wait()
            wh[cb] = pltpu.async_copy(
                bufs[cb], out_ref.at[didx_v.at[c]], wsems[cb])
        for h in wh:
            if h is not None:
                h.wait()

    return pl.kernel(
        body,
        out_type=jax.ShapeDtypeStruct((n_out, _D), table.dtype),
        mesh=mesh,
        scratch_types=[
            pltpu.VMEM((chunks, _GC), jnp.int32),
            pltpu.VMEM((chunks, _GC), jnp.int32),
            pltpu.VMEM((_GC, _D), jnp.float32),
            pltpu.VMEM((_GC, _D), jnp.float32),
            pltpu.SemaphoreType.DMA,
            pltpu.SemaphoreType.DMA,
            pltpu.SemaphoreType.DMA,
            pltpu.SemaphoreType.DMA,
        ],
    )(table, src_idx.reshape(_NW * chunks, _GC),
      dst_idx.reshape(_NW * chunks, _GC))


def _mm_body(gid_ref, x_ref, w_ref, b_ref, o_ref):
    i = pl.program_id(0)

    @pl.when(gid_ref[i] < 64)
    def _():
        xb = x_ref[...].astype(jnp.bfloat16)
        wb = w_ref[0].astype(jnp.bfloat16)
        o_ref[...] = (
            jnp.dot(xb, wb, preferred_element_type=jnp.float32) + b_ref[0]
        )


def _segment_matmul(xs, weight, bias, tile_gid):
    n_tiles = xs.shape[0] // _BN
    last = n_tiles - 1

    grid_spec = pltpu.PrefetchScalarGridSpec(
        num_scalar_prefetch=1,
        grid=(n_tiles,),
        in_specs=[
            pl.BlockSpec(
                (_BN, _D), lambda i, g: (jnp.where(g[i] < 64, i, last), 0)),
            pl.BlockSpec((1, _D, _D), lambda i, g: (g[i] & 63, 0, 0)),
            pl.BlockSpec((1, 1, _D), lambda i, g: (g[i] & 63, 0, 0)),
        ],
        out_specs=pl.BlockSpec(
            (_BN, _D), lambda i, g: (jnp.where(g[i] < 64, i, last), 0)),
    )
    return pl.pallas_call(
        _mm_body,
        grid_spec=grid_spec,
        out_shape=jax.ShapeDtypeStruct((xs.shape[0], _D), jnp.float32),
    )(tile_gid, xs, weight, bias[:, None, :])


def kernel(x, type_vec, weight, bias):
    n = x.shape[0]
    t = weight.shape[0]
    n_pad = n + t * _BN

    counts = jnp.zeros((t,), jnp.int32).at[type_vec].add(1)
    group_start = jnp.cumsum(counts) - counts
    padded_counts = ((counts + _BN - 1) // _BN) * _BN
    padded_start = jnp.cumsum(padded_counts) - padded_counts
    perm = jnp.argsort(type_vec).astype(jnp.int32)
    sorted_types = type_vec[perm]
    delta = padded_start - group_start
    ppos = jnp.arange(n, dtype=jnp.int32) + delta[sorted_types]
    n_tiles = n_pad // _BN
    tile_base = jnp.arange(n_tiles, dtype=jnp.int32) * _BN
    tile_gid = (
        jnp.sum(padded_start[None, :] <= tile_base[:, None], axis=1,
                dtype=jnp.int32) - 1
    )
    occupied_tiles = jnp.sum(padded_counts, dtype=jnp.int32) // _BN
    tile_gid = tile_gid + 64 * (
        jnp.arange(n_tiles, dtype=jnp.int32) >= occupied_tiles
    ).astype(jnp.int32)

    xs = _sc_permute(x, perm, ppos, n, n_pad)
    ys = _segment_matmul(xs, weight, bias, tile_gid)
    return _sc_permute(ys, ppos, perm, n, n)
